# Initial kernel scaffold; baseline (speedup 1.0000x reference)
import jax
import jax.numpy as jnp
from jax.experimental import pallas as pl


def kernel(features, adj0, adj1, W_0_0, b_0_0, W_0_1, b_0_1, W_1_0, b_1_0, W_1_1, b_1_1):
    raise NotImplementedError("write your pallas kernel here")


# speedup vs baseline: 12.7001x; 12.7001x over previous
import functools

import numpy as np
import jax
import jax.numpy as jnp
from jax.experimental import pallas as pl

_NEG = np.float32(-1e9)
_TOPBIT = np.int32(-(2 ** 31))
_LOW31 = np.int32(0x7FFFFFFF)


def _hn_body(x_ref, w0_ref, b0_ref, w1_ref, b1_ref, hn0_ref, hn1_ref):
    x = x_ref[...]
    for w_ref, b_ref, o_ref in ((w0_ref, b0_ref, hn0_ref),
                                (w1_ref, b1_ref, hn1_ref)):
        h = jnp.dot(x, w_ref[...], preferred_element_type=jnp.float32) + b_ref[...]
        nrm = jnp.sqrt(jnp.sum(h * h, axis=-1, keepdims=True))
        o_ref[...] = h / (nrm + jnp.float32(1e-8))


def _ordered_key(bits):
    return bits ^ (jax.lax.shift_right_arithmetic(bits, 31) & _LOW31)


def _layer_body(adj_ref, hn0_ref, hn1_ref, x_ref, out_ref, *, br, k):
    i = pl.program_id(0)
    adj = adj_ref[...]
    x = x_ref[...]
    acc = None
    for hn_ref in (hn0_ref, hn1_ref):
        hnf = hn_ref[...]
        hnb = hn_ref[pl.ds(i * br, br), :]
        sim = jax.lax.dot_general(hnb, hnf, (((1,), (1,)), ((), ())),
                                  preferred_element_type=jnp.float32)
        scores = jnp.where(adj > 0, sim, _NEG)
        skey = _ordered_key(jax.lax.bitcast_convert_type(scores, jnp.int32))
        acc_b = jnp.zeros((br, 1), jnp.int32)
        for bit in range(31, -1, -1):
            cand_b = acc_b | jnp.int32(int(np.int32(np.uint32(1 << bit))))
            cand_s = cand_b ^ _TOPBIT
            cnt = jnp.sum((skey >= cand_s).astype(jnp.int32),
                          axis=-1, keepdims=True)
            acc_b = jnp.where(cnt >= k, cand_b, acc_b)
        vk = jax.lax.bitcast_convert_type(_ordered_key(acc_b ^ _TOPBIT),
                                          jnp.float32)
        mask = scores >= vk
        m = jnp.max(scores, axis=-1, keepdims=True)
        p = jnp.where(mask, jnp.exp(scores - m), jnp.float32(0.0))
        s = jnp.sum(p, axis=-1, keepdims=True)
        attn = p / s
        o = jax.lax.dot_general(attn, x, (((1,), (0,)), ((), ())),
                                preferred_element_type=jnp.float32)
        acc = o if acc is None else acc + o
    out_ref[...] = acc * jnp.float32(0.5)


def _layer(x, adj, W0, b0, W1, b1, br):
    n, d = x.shape
    hid = W0.shape[1]
    hn0, hn1 = pl.pallas_call(
        _hn_body,
        out_shape=[jax.ShapeDtypeStruct((n, hid), jnp.float32)] * 2,
    )(x, W0, b0.reshape(1, hid), W1, b1.reshape(1, hid))
    k = max(1, int(0.5 * n))
    out = pl.pallas_call(
        functools.partial(_layer_body, br=br, k=k),
        grid=(n // br,),
        in_specs=[
            pl.BlockSpec((br, n), lambda i: (i, 0)),
            pl.BlockSpec((n, hid), lambda i: (0, 0)),
            pl.BlockSpec((n, hid), lambda i: (0, 0)),
            pl.BlockSpec((n, d), lambda i: (0, 0)),
        ],
        out_specs=pl.BlockSpec((br, d), lambda i: (i, 0)),
        out_shape=jax.ShapeDtypeStruct((n, d), jnp.float32),
    )(adj, hn0, hn1, x)
    return out


def kernel(features, adj0, adj1, W_0_0, b_0_0, W_0_1, b_0_1,
           W_1_0, b_1_0, W_1_1, b_1_1):
    x = _layer(features, adj0, W_0_0, b_0_0, W_0_1, b_0_1, 256)
    x = _layer(x, adj1, W_1_0, b_1_0, W_1_1, b_1_1, 256)
    return x


# speedup vs baseline: 17.2225x; 1.3561x over previous
import functools

import numpy as np
import jax
import jax.numpy as jnp
from jax.experimental import pallas as pl

_NEG = np.float32(-1e9)
_TOPBIT = np.int32(-(2 ** 31))
_LOW31 = np.int32(0x7FFFFFFF)


def _hn_body(x_ref, w0_ref, b0_ref, w1_ref, b1_ref, hn0_ref, hn1_ref):
    x = x_ref[...]
    for w_ref, b_ref, o_ref in ((w0_ref, b0_ref, hn0_ref),
                                (w1_ref, b1_ref, hn1_ref)):
        h = jnp.dot(x, w_ref[...], preferred_element_type=jnp.float32) + b_ref[...]
        nrm = jnp.sqrt(jnp.sum(h * h, axis=-1, keepdims=True))
        o_ref[...] = h / (nrm + jnp.float32(1e-8))


def _ordered_key(bits):
    return bits ^ (jax.lax.shift_right_arithmetic(bits, 31) & _LOW31)


def _layer_body(adj_ref, hn0_ref, hn1_ref, x_ref, out_ref, *, br, k):
    i = pl.program_id(0)
    adj = adj_ref[...]
    x = x_ref[...]
    acc = None
    for hn_ref in (hn0_ref, hn1_ref):
        hnf = hn_ref[...]
        hnb = hn_ref[pl.ds(i * br, br), :]
        sim = jax.lax.dot_general(hnb, hnf, (((1,), (1,)), ((), ())),
                                  preferred_element_type=jnp.float32)
        scores = jnp.where(adj > 0, sim, _NEG)
        mdom = jnp.maximum(scores + jnp.float32(6.0), jnp.float32(4.0))
        skey = jax.lax.bitcast_convert_type(mdom, jnp.int32)
        acc_b = jnp.full((br, 1), np.int32(0x40800000), jnp.int32)
        for bit in range(22, -1, -1):
            cand = acc_b | np.int32(1 << bit)
            cnt = jnp.sum((skey >= cand).astype(jnp.int32),
                          axis=-1, keepdims=True)
            acc_b = jnp.where(cnt >= k, cand, acc_b)
        vt = jax.lax.bitcast_convert_type(acc_b, jnp.float32)
        mask = mdom >= vt
        m = jnp.max(scores, axis=-1, keepdims=True)
        p = jnp.where(mask, jnp.exp(scores - m), jnp.float32(0.0))
        s = jnp.sum(p, axis=-1, keepdims=True)
        attn = p / s
        o = jax.lax.dot_general(attn, x, (((1,), (0,)), ((), ())),
                                preferred_element_type=jnp.float32)
        acc = o if acc is None else acc + o
    out_ref[...] = acc * jnp.float32(0.5)


def _layer(x, adj, W0, b0, W1, b1, br):
    n, d = x.shape
    hid = W0.shape[1]
    hn0, hn1 = pl.pallas_call(
        _hn_body,
        out_shape=[jax.ShapeDtypeStruct((n, hid), jnp.float32)] * 2,
    )(x, W0, b0.reshape(1, hid), W1, b1.reshape(1, hid))
    k = max(1, int(0.5 * n))
    out = pl.pallas_call(
        functools.partial(_layer_body, br=br, k=k),
        grid=(n // br,),
        in_specs=[
            pl.BlockSpec((br, n), lambda i: (i, 0)),
            pl.BlockSpec((n, hid), lambda i: (0, 0)),
            pl.BlockSpec((n, hid), lambda i: (0, 0)),
            pl.BlockSpec((n, d), lambda i: (0, 0)),
        ],
        out_specs=pl.BlockSpec((br, d), lambda i: (i, 0)),
        out_shape=jax.ShapeDtypeStruct((n, d), jnp.float32),
    )(adj, hn0, hn1, x)
    return out


def kernel(features, adj0, adj1, W_0_0, b_0_0, W_0_1, b_0_1,
           W_1_0, b_1_0, W_1_1, b_1_1):
    x = _layer(features, adj0, W_0_0, b_0_0, W_0_1, b_0_1, 256)
    x = _layer(x, adj1, W_1_0, b_1_0, W_1_1, b_1_1, 256)
    return x
